# baseline (speedup 1.0000x reference)
import jax
import jax.numpy as jnp
from jax import lax
from jax.experimental import pallas as pl
from jax.experimental.pallas import tpu as pltpu
from jax.experimental.pallas import tpu_sc as plsc

N = 10000
E = 320000
D = 128
H = 64

NC = 2
NS = 16
NW = NC * NS
L = 16

NPAD = 10240
SLICE = NPAD // NS
EPW = E // NW
CH = 80
NCHUNK = EPW // CH
NSUB = 2
SUB = 40
CHB = SUB * NSUB
NBCH = EPW // CHB

_MESH = plsc.VectorSubcoreMesh(core_axis_name="c", subcore_axis_name="s")
_SC_PARAMS = pltpu.CompilerParams(needs_layout_passes=False,
                                  use_tc_tiling_on_sc=False)

ROWB = 1000
GRID = N // ROWB


def _f32(*shape):
    return jax.ShapeDtypeStruct(shape, jnp.float32)


def _tc_encode_body(x_ref, w_ref, a_ref, h_ref, sd_ref):
    h = jnp.dot(x_ref[...], w_ref[...], preferred_element_type=jnp.float32)
    h_ref[...] = h
    sd_ref[...] = jnp.dot(h, a_ref[...], preferred_element_type=jnp.float32)


def _tc_encode(x, w, a):
    din = x.shape[1]
    return pl.pallas_call(
        _tc_encode_body,
        grid=(GRID,),
        in_specs=[
            pl.BlockSpec((ROWB, din), lambda i: (i, 0)),
            pl.BlockSpec((din, H), lambda i: (0, 0)),
            pl.BlockSpec((H, 8), lambda i: (0, 0)),
        ],
        out_specs=[
            pl.BlockSpec((ROWB, H), lambda i: (i, 0)),
            pl.BlockSpec((ROWB, 8), lambda i: (i, 0)),
        ],
        out_shape=[_f32(N, H), _f32(N, 8)],
    )(x, w, a)


def _norm_z(o0, o1, d0, d1, b):
    return jnp.maximum((o0 + o1) / (d0 + d1 + 1e-16) + b, 0.0)


def _tc_mid_body(o0_ref, o1_ref, d0_ref, d1_ref, b_ref, w_ref, a_ref,
                 h_ref, sd_ref):
    z = _norm_z(o0_ref[0], o1_ref[0], d0_ref[...], d1_ref[...], b_ref[...])
    h = jnp.dot(z, w_ref[...], preferred_element_type=jnp.float32)
    h_ref[...] = h
    sd_ref[...] = jnp.dot(h, a_ref[...], preferred_element_type=jnp.float32)


def _tc_mid(op, d0, d1, b, w, a):
    return pl.pallas_call(
        _tc_mid_body,
        grid=(GRID,),
        in_specs=[
            pl.BlockSpec((1, ROWB, H), lambda i: (0, i, 0)),
            pl.BlockSpec((1, ROWB, H), lambda i: (1, i, 0)),
            pl.BlockSpec((ROWB, 1), lambda i: (i, 0)),
            pl.BlockSpec((ROWB, 1), lambda i: (i, 0)),
            pl.BlockSpec((1, H), lambda i: (0, 0)),
            pl.BlockSpec((H, H), lambda i: (0, 0)),
            pl.BlockSpec((H, 8), lambda i: (0, 0)),
        ],
        out_specs=[
            pl.BlockSpec((ROWB, H), lambda i: (i, 0)),
            pl.BlockSpec((ROWB, 8), lambda i: (i, 0)),
        ],
        out_shape=[_f32(N, H), _f32(N, 8)],
    )(op, op, d0, d1, b, w, a)


def _tc_head_body(o0_ref, o1_ref, d0_ref, d1_ref, b_ref, nw1_ref, nb1_ref,
                  nw2_ref, nb2_ref, ea_ref, eb1_ref, eb_ref,
                  nf_ref, p_ref, q_ref):
    z = _norm_z(o0_ref[0], o1_ref[0], d0_ref[...], d1_ref[...], b_ref[...])
    t = jnp.maximum(
        jnp.dot(z, nw1_ref[...], preferred_element_type=jnp.float32)
        + nb1_ref[...], 0.0)
    nf_ref[...] = (jnp.dot(t, nw2_ref[...], preferred_element_type=jnp.float32)
                   + nb2_ref[...])
    p_ref[...] = (jnp.dot(z, ea_ref[...], preferred_element_type=jnp.float32)
                  + eb1_ref[...]).astype(jnp.bfloat16)
    q_ref[...] = jnp.dot(z, eb_ref[...],
                         preferred_element_type=jnp.float32).astype(jnp.bfloat16)


def _tc_head(op, d0, d1, b, nw1, nb1, nw2, nb2, ea, eb1, eb):
    return pl.pallas_call(
        _tc_head_body,
        grid=(GRID,),
        in_specs=[
            pl.BlockSpec((1, ROWB, H), lambda i: (0, i, 0)),
            pl.BlockSpec((1, ROWB, H), lambda i: (1, i, 0)),
            pl.BlockSpec((ROWB, 1), lambda i: (i, 0)),
            pl.BlockSpec((ROWB, 1), lambda i: (i, 0)),
            pl.BlockSpec((1, H), lambda i: (0, 0)),
            pl.BlockSpec((H, H), lambda i: (0, 0)),
            pl.BlockSpec((1, H), lambda i: (0, 0)),
            pl.BlockSpec((H, D), lambda i: (0, 0)),
            pl.BlockSpec((1, D), lambda i: (0, 0)),
            pl.BlockSpec((H, H), lambda i: (0, 0)),
            pl.BlockSpec((1, H), lambda i: (0, 0)),
            pl.BlockSpec((H, H), lambda i: (0, 0)),
        ],
        out_specs=[
            pl.BlockSpec((ROWB, D), lambda i: (i, 0)),
            pl.BlockSpec((ROWB, H), lambda i: (i, 0)),
            pl.BlockSpec((ROWB, H), lambda i: (i, 0)),
        ],
        out_shape=[_f32(N, D),
                   jax.ShapeDtypeStruct((N, H), jnp.bfloat16),
                   jax.ShapeDtypeStruct((N, H), jnp.bfloat16)],
    )(op, op, d0, d1, b, nw1, nb1, nw2, nb2, ea, eb1, eb)


VSR = E // 8
EPB = 1000


def _tc_epfin_body(vs_ref, sel_ref, b2_ref, out_ref):
    acc = jnp.dot(vs_ref[...], sel_ref[...], preferred_element_type=jnp.float32)
    out_ref[...] = jax.nn.sigmoid(acc + b2_ref[...])


def _tc_epfin(vs, sel, b2):
    return pl.pallas_call(
        _tc_epfin_body,
        grid=(VSR // EPB,),
        in_specs=[
            pl.BlockSpec((EPB, 128), lambda i: (i, 0)),
            pl.BlockSpec((128, 8), lambda i: (0, 0)),
            pl.BlockSpec((1, 1), lambda i: (0, 0)),
        ],
        out_specs=pl.BlockSpec((EPB, 8), lambda i: (i, 0)),
        out_shape=_f32(VSR, 8),
    )(vs, sel, b2)


def _make_sc_layer():
    def body(h_hbm, s_hbm, d_hbm, esrc_hbm, edst_hbm, denp_hbm, outp_hbm,
             s_v, d_v, src_v, dst_v, den_v, red_v, acc_v, ex_all,
             d80_0, d80_1, rows_0, rows_1, den_sh, out_sh,
             gsem_0, gsem_1, ssem_0, ssem_1):
        c = lax.axis_index("c")
        sid = lax.axis_index("s")
        w = sid * NC + c
        base = w * EPW

        pltpu.sync_copy(s_hbm, s_v)
        pltpu.sync_copy(d_hbm, d_v)
        pltpu.sync_copy(esrc_hbm.at[pl.ds(base, EPW)], src_v)
        pltpu.sync_copy(edst_hbm.at[pl.ds(base, EPW)], dst_v)

        def zero_body(g, _):
            den_v[pl.ds(g * L, L)] = jnp.zeros((L,), jnp.float32)
            return 0
        lax.fori_loop(0, NPAD // L, zero_body, 0)

        def zz_body(r, _):
            for t in range(H // L):
                rows_0[r, pl.ds(t * L, L)] = jnp.zeros((L,), jnp.float32)
            return 0
        lax.fori_loop(0, CHB, zz_body, 0)
        for q in range(SLICE // CHB):
            pltpu.sync_copy(
                rows_0, out_sh.at[pl.ds(sid * SLICE + q * CHB, CHB), :])
        for t in range(SLICE % CHB // 8):
            pass
        if SLICE % CHB:
            pltpu.sync_copy(
                rows_0.at[pl.ds(0, SLICE % CHB), :],
                out_sh.at[pl.ds(sid * SLICE + (SLICE // CHB) * CHB,
                                SLICE % CHB), :])
        plsc.subcore_barrier()

        def start_gather(k, rows_b, gsem_b):
            for j in range(NSUB):
                pltpu.async_copy(
                    h_hbm.at[src_v.at[pl.ds(k * CHB + j * SUB, SUB)]],
                    rows_b.at[pl.ds(j * SUB, SUB), :], gsem_b)

        def process(k, d80_b, rows_b, gsem_b, ssem_b):
            def d_body(g, _):
                j = lax.div(g * L, SUB)
                off = lax.rem(g * L, SUB)
                sl = pl.ds(k * CHB + g * L, L)
                s16 = src_v[sl]
                d16 = dst_v[sl]
                e = plsc.load_gather(s_v, [s16]) + plsc.load_gather(d_v, [d16])
                e = jnp.where(e > 0, e, e * 0.2)
                ex = jnp.exp(jnp.minimum(e, 60.0))
                ex_all[pl.ds(g * L, L)] = ex
                plsc.addupdate_scatter(den_v, [d16], ex)
                d80_b[j, pl.ds(off, L)] = d16
                return 0
            lax.fori_loop(0, CHB // L, d_body, 0)
            for j in range(NSUB):
                pltpu.make_async_copy(
                    h_hbm.at[src_v.at[pl.ds(k * CHB + j * SUB, SUB)]],
                    rows_b.at[pl.ds(j * SUB, SUB), :], gsem_b).wait()

            def sc_body(g, _):
                for j in range(L):
                    av = plsc.load_gather(
                        ex_all, [jnp.full((L,), g * L + j, jnp.int32)])
                    r = g * L + j
                    for t in range(H // L):
                        rows_b[r, pl.ds(t * L, L)] = (
                            rows_b[r, pl.ds(t * L, L)] * av)
                return 0
            lax.fori_loop(0, CHB // L, sc_body, 0)
            for j in range(NSUB):
                pltpu.async_copy(rows_b.at[pl.ds(j * SUB, SUB), :],
                                 out_sh.at[d80_b.at[j]], ssem_b, add=True)

        def wait_scatter(rows_b, d80_b, ssem_b):
            for j in range(NSUB):
                pltpu.make_async_copy(rows_b.at[pl.ds(j * SUB, SUB), :],
                                      out_sh.at[d80_b.at[j]], ssem_b).wait()

        start_gather(0, rows_0, gsem_0)

        def pair_body(p, _):
            k0 = p * 2
            k1 = k0 + 1

            @pl.when(p > 0)
            def _():
                wait_scatter(rows_1, d80_1, ssem_1)
            start_gather(k1, rows_1, gsem_1)
            process(k0, d80_0, rows_0, gsem_0, ssem_0)
            wait_scatter(rows_0, d80_0, ssem_0)

            @pl.when(k1 + 1 < NBCH)
            def _():
                start_gather(k1 + 1, rows_0, gsem_0)
            process(k1, d80_1, rows_1, gsem_1, ssem_1)
            return 0
        lax.fori_loop(0, NBCH // 2, pair_body, 0)
        if NBCH % 2:
            process(NBCH - 1, d80_0, rows_0, gsem_0, ssem_0)
            wait_scatter(rows_0, d80_0, ssem_0)
            wait_scatter(rows_1, d80_1, ssem_1)
        else:
            wait_scatter(rows_1, d80_1, ssem_1)

        plsc.subcore_barrier()

        pltpu.sync_copy(den_v, den_sh.at[sid])
        plsc.subcore_barrier()
        cbase = sid * SLICE

        pltpu.sync_copy(den_sh.at[:, pl.ds(cbase, SLICE)], red_v)

        def red_body(g, _):
            sl = pl.ds(g * L, L)
            acc = red_v[0, sl]
            for t in range(1, NS):
                acc = acc + red_v[t, sl]
            acc_v[sl] = acc
            return 0
        lax.fori_loop(0, SLICE // L, red_body, 0)

        pltpu.sync_copy(acc_v, denp_hbm.at[c, pl.ds(cbase, SLICE)])
        pltpu.sync_copy(out_sh.at[pl.ds(sid * SLICE, SLICE), :],
                        outp_hbm.at[c, pl.ds(sid * SLICE, SLICE), :])

    return pl.kernel(
        body,
        out_type=[_f32(NC, NPAD), _f32(NC, NPAD, H)],
        mesh=_MESH,
        compiler_params=_SC_PARAMS,
        scratch_types=[
            pltpu.VMEM((N,), jnp.float32),
            pltpu.VMEM((N,), jnp.float32),
            pltpu.VMEM((EPW,), jnp.int32),
            pltpu.VMEM((EPW,), jnp.int32),
            pltpu.VMEM((NPAD,), jnp.float32),
            pltpu.VMEM((NS, SLICE), jnp.float32),
            pltpu.VMEM((SLICE,), jnp.float32),
            pltpu.VMEM((CHB,), jnp.float32),
            pltpu.VMEM((NSUB, SUB), jnp.int32),
            pltpu.VMEM((NSUB, SUB), jnp.int32),
            pltpu.VMEM((CHB, H), jnp.float32),
            pltpu.VMEM((CHB, H), jnp.float32),
            pltpu.VMEM_SHARED((NS, NPAD), jnp.float32),
            pltpu.VMEM_SHARED((NPAD, H), jnp.float32),
            pltpu.SemaphoreType.DMA,
            pltpu.SemaphoreType.DMA,
            pltpu.SemaphoreType.DMA,
            pltpu.SemaphoreType.DMA,
        ],
    )


_sc_layer = _make_sc_layer()


def _make_sc_ep():
    def body(p_hbm, q_hbm, esrc_hbm, edst_hbm, w2_hbm, vs_hbm,
             src_v, dst_v, w2_v, pr_0, qr_0, pr_1, qr_1, vsb_0, vsb_1,
             psem_0, qsem_0, psem_1, qsem_1, wsem_0, wsem_1):
        c = lax.axis_index("c")
        sid = lax.axis_index("s")
        w = sid * NC + c
        base = w * EPW
        vbase = w * (EPW // 8)

        pltpu.sync_copy(w2_hbm, w2_v)
        pltpu.sync_copy(esrc_hbm.at[pl.ds(base, EPW)], src_v)
        pltpu.sync_copy(edst_hbm.at[pl.ds(base, EPW)], dst_v)
        w2u = []
        for t in range(2):
            wa, wb = plsc.unpack(w2_v[pl.ds(t * 2 * L, 2 * L)],
                                 format=plsc.PackFormat.INTERLEAVED)
            w2u += [wa, wb]

        def start_gather(k, pr_b, qr_b, psem_b, qsem_b):
            pltpu.async_copy(p_hbm.at[src_v.at[pl.ds(k * CH, CH)]], pr_b,
                             psem_b)
            pltpu.async_copy(q_hbm.at[dst_v.at[pl.ds(k * CH, CH)]], qr_b,
                             qsem_b)

        def process(k, pr_b, qr_b, psem_b, qsem_b, vsb_b, wsem_b):
            pltpu.make_async_copy(
                p_hbm.at[src_v.at[pl.ds(k * CH, CH)]], pr_b, psem_b).wait()
            pltpu.make_async_copy(
                q_hbm.at[dst_v.at[pl.ds(k * CH, CH)]], qr_b, qsem_b).wait()
            def dot_body(i, _):
                acc = jnp.zeros((L,), jnp.float32)
                for t in range(2):
                    sl = pl.ds(t * 2 * L, 2 * L)
                    pa, pb = plsc.unpack(pr_b[i, sl],
                                         format=plsc.PackFormat.INTERLEAVED)
                    qa, qb = plsc.unpack(qr_b[i, sl],
                                         format=plsc.PackFormat.INTERLEAVED)
                    acc = acc + jnp.maximum(pa + qa, 0.0) * w2u[2 * t]
                    acc = acc + jnp.maximum(pb + qb, 0.0) * w2u[2 * t + 1]
                vsb_b[lax.div(i, 8), pl.ds(lax.rem(i, 8) * L, L)] = acc
                return 0
            lax.fori_loop(0, CH, dot_body, 0)
            pltpu.async_copy(
                vsb_b, vs_hbm.at[pl.ds(vbase + k * (CH // 8), CH // 8), :],
                wsem_b)

        def wait_write(k, vsb_b, wsem_b):
            pltpu.make_async_copy(
                vsb_b, vs_hbm.at[pl.ds(vbase + k * (CH // 8), CH // 8), :],
                wsem_b).wait()

        start_gather(0, pr_0, qr_0, psem_0, qsem_0)

        def pair_body(p, _):
            k0 = p * 2
            k1 = k0 + 1
            start_gather(k1, pr_1, qr_1, psem_1, qsem_1)

            @pl.when(p > 0)
            def _():
                wait_write(k0 - 2, vsb_0, wsem_0)
            process(k0, pr_0, qr_0, psem_0, qsem_0, vsb_0, wsem_0)

            @pl.when(k1 + 1 < NCHUNK)
            def _():
                start_gather(k1 + 1, pr_0, qr_0, psem_0, qsem_0)

            @pl.when(p > 0)
            def _():
                wait_write(k1 - 2, vsb_1, wsem_1)
            process(k1, pr_1, qr_1, psem_1, qsem_1, vsb_1, wsem_1)
            return 0
        lax.fori_loop(0, NCHUNK // 2, pair_body, 0)
        if NCHUNK % 2:
            k = NCHUNK - 1
            wait_write(k - 2, vsb_0, wsem_0)
            process(k, pr_0, qr_0, psem_0, qsem_0, vsb_0, wsem_0)
            wait_write(k, vsb_0, wsem_0)
            wait_write(k - 1, vsb_1, wsem_1)
        else:
            wait_write(NCHUNK - 2, vsb_0, wsem_0)
            wait_write(NCHUNK - 1, vsb_1, wsem_1)

    return pl.kernel(
        body,
        out_type=[_f32(VSR, 128)],
        mesh=_MESH,
        compiler_params=_SC_PARAMS,
        scratch_types=[
            pltpu.VMEM((EPW,), jnp.int32),
            pltpu.VMEM((EPW,), jnp.int32),
            pltpu.VMEM((H,), jnp.bfloat16),
            pltpu.VMEM((CH, H), jnp.bfloat16),
            pltpu.VMEM((CH, H), jnp.bfloat16),
            pltpu.VMEM((CH, H), jnp.bfloat16),
            pltpu.VMEM((CH, H), jnp.bfloat16),
            pltpu.VMEM((CH // 8, 128), jnp.float32),
            pltpu.VMEM((CH // 8, 128), jnp.float32),
            pltpu.SemaphoreType.DMA,
            pltpu.SemaphoreType.DMA,
            pltpu.SemaphoreType.DMA,
            pltpu.SemaphoreType.DMA,
            pltpu.SemaphoreType.DMA,
            pltpu.SemaphoreType.DMA,
        ],
    )


_sc_ep = _make_sc_ep()


def kernel(x, edge_index, W1, as1, ad1, b1, W2, as2, ad2, b2,
           W3, as3, ad3, b3, npW1, npb1, npW2, npb2, epW1, epb1, epW2, epb2):
    zpad = jnp.zeros((H, 6), jnp.float32)
    A1 = jnp.concatenate([as1[:, None], ad1[:, None], zpad], axis=1)
    A2 = jnp.concatenate([as2[:, None], ad2[:, None], zpad], axis=1)
    A3 = jnp.concatenate([as3[:, None], ad3[:, None], zpad], axis=1)

    esrc = edge_index[0]
    edst = edge_index[1]

    h1, sd1 = _tc_encode(x, W1, A1)
    denp1, outp1 = _sc_layer(h1, sd1[:, 0], sd1[:, 1], esrc, edst)

    h2, sd2 = _tc_mid(outp1, denp1[0][:, None], denp1[1][:, None],
                      b1[None, :], W2, A2)
    denp2, outp2 = _sc_layer(h2, sd2[:, 0], sd2[:, 1], esrc, edst)

    h3, sd3 = _tc_mid(outp2, denp2[0][:, None], denp2[1][:, None],
                      b2[None, :], W3, A3)
    denp3, outp3 = _sc_layer(h3, sd3[:, 0], sd3[:, 1], esrc, edst)

    nf, P, Q = _tc_head(outp3, denp3[0][:, None],
                        denp3[1][:, None], b3[None, :],
                        npW1, npb1[None, :], npW2, npb2[None, :],
                        epW1[:H], epb1[None, :], epW1[H:])

    (vs,) = _sc_ep(P, Q, esrc, edst, epW2[:, 0].astype(jnp.bfloat16))

    lane = jnp.arange(128) // 16
    grp = jnp.arange(8)
    sel = (lane[:, None] == grp[None, :]).astype(jnp.float32)
    ep8 = _tc_epfin(vs, sel, epb2[None, :])
    return nf, ep8.reshape(E, 1)


# speedup vs baseline: 1.3921x; 1.3921x over previous
import jax
import jax.numpy as jnp
from jax import lax
from jax.experimental import pallas as pl
from jax.experimental.pallas import tpu as pltpu
from jax.experimental.pallas import tpu_sc as plsc

N = 10000
E = 320000
D = 128
H = 64

NC = 2
NS = 16
NW = NC * NS
L = 16

NPAD = 10240
SLICE = NPAD // NS
EPW = E // NW
CH = 80
NCHUNK = EPW // CH
NSUB = 2
SUB = 40
CHB = SUB * NSUB
NBCH = EPW // CHB

_MESH = plsc.VectorSubcoreMesh(core_axis_name="c", subcore_axis_name="s")
_SC_PARAMS = pltpu.CompilerParams(needs_layout_passes=False,
                                  use_tc_tiling_on_sc=False)

ROWB = 1000
GRID = N // ROWB


def _f32(*shape):
    return jax.ShapeDtypeStruct(shape, jnp.float32)


def _tc_encode_body(x_ref, w_ref, a_ref, h_ref, sd_ref):
    h = jnp.dot(x_ref[...], w_ref[...], preferred_element_type=jnp.float32)
    h_ref[...] = h
    sd_ref[...] = jnp.dot(h, a_ref[...], preferred_element_type=jnp.float32)


def _tc_encode(x, w, a):
    din = x.shape[1]
    return pl.pallas_call(
        _tc_encode_body,
        grid=(GRID,),
        in_specs=[
            pl.BlockSpec((ROWB, din), lambda i: (i, 0)),
            pl.BlockSpec((din, H), lambda i: (0, 0)),
            pl.BlockSpec((H, 8), lambda i: (0, 0)),
        ],
        out_specs=[
            pl.BlockSpec((ROWB, H), lambda i: (i, 0)),
            pl.BlockSpec((ROWB, 8), lambda i: (i, 0)),
        ],
        out_shape=[_f32(N, H), _f32(N, 8)],
    )(x, w, a)


def _norm_z(o0, o1, d0, d1, b):
    return jnp.maximum((o0 + o1) / (d0 + d1 + 1e-16) + b, 0.0)


def _tc_mid_body(o0_ref, o1_ref, d0_ref, d1_ref, b_ref, w_ref, a_ref,
                 h_ref, sd_ref):
    z = _norm_z(o0_ref[0], o1_ref[0], d0_ref[...], d1_ref[...], b_ref[...])
    h = jnp.dot(z, w_ref[...], preferred_element_type=jnp.float32)
    h_ref[...] = h
    sd_ref[...] = jnp.dot(h, a_ref[...], preferred_element_type=jnp.float32)


def _tc_mid(op, d0, d1, b, w, a):
    return pl.pallas_call(
        _tc_mid_body,
        grid=(GRID,),
        in_specs=[
            pl.BlockSpec((1, ROWB, H), lambda i: (0, i, 0)),
            pl.BlockSpec((1, ROWB, H), lambda i: (1, i, 0)),
            pl.BlockSpec((ROWB, 1), lambda i: (i, 0)),
            pl.BlockSpec((ROWB, 1), lambda i: (i, 0)),
            pl.BlockSpec((1, H), lambda i: (0, 0)),
            pl.BlockSpec((H, H), lambda i: (0, 0)),
            pl.BlockSpec((H, 8), lambda i: (0, 0)),
        ],
        out_specs=[
            pl.BlockSpec((ROWB, H), lambda i: (i, 0)),
            pl.BlockSpec((ROWB, 8), lambda i: (i, 0)),
        ],
        out_shape=[_f32(N, H), _f32(N, 8)],
    )(op, op, d0, d1, b, w, a)


def _tc_head_body(o0_ref, o1_ref, d0_ref, d1_ref, b_ref, nw1_ref, nb1_ref,
                  nw2_ref, nb2_ref, ea_ref, eb1_ref, eb_ref,
                  nf_ref, p_ref, q_ref):
    z = _norm_z(o0_ref[0], o1_ref[0], d0_ref[...], d1_ref[...], b_ref[...])
    t = jnp.maximum(
        jnp.dot(z, nw1_ref[...], preferred_element_type=jnp.float32)
        + nb1_ref[...], 0.0)
    nf_ref[...] = (jnp.dot(t, nw2_ref[...], preferred_element_type=jnp.float32)
                   + nb2_ref[...])
    p_ref[...] = (jnp.dot(z, ea_ref[...], preferred_element_type=jnp.float32)
                  + eb1_ref[...]).astype(jnp.bfloat16)
    q_ref[...] = jnp.dot(z, eb_ref[...],
                         preferred_element_type=jnp.float32).astype(jnp.bfloat16)


def _tc_head(op, d0, d1, b, nw1, nb1, nw2, nb2, ea, eb1, eb):
    return pl.pallas_call(
        _tc_head_body,
        grid=(GRID,),
        in_specs=[
            pl.BlockSpec((1, ROWB, H), lambda i: (0, i, 0)),
            pl.BlockSpec((1, ROWB, H), lambda i: (1, i, 0)),
            pl.BlockSpec((ROWB, 1), lambda i: (i, 0)),
            pl.BlockSpec((ROWB, 1), lambda i: (i, 0)),
            pl.BlockSpec((1, H), lambda i: (0, 0)),
            pl.BlockSpec((H, H), lambda i: (0, 0)),
            pl.BlockSpec((1, H), lambda i: (0, 0)),
            pl.BlockSpec((H, D), lambda i: (0, 0)),
            pl.BlockSpec((1, D), lambda i: (0, 0)),
            pl.BlockSpec((H, H), lambda i: (0, 0)),
            pl.BlockSpec((1, H), lambda i: (0, 0)),
            pl.BlockSpec((H, H), lambda i: (0, 0)),
        ],
        out_specs=[
            pl.BlockSpec((ROWB, D), lambda i: (i, 0)),
            pl.BlockSpec((ROWB, H), lambda i: (i, 0)),
            pl.BlockSpec((ROWB, H), lambda i: (i, 0)),
        ],
        out_shape=[_f32(N, D),
                   jax.ShapeDtypeStruct((N, H), jnp.bfloat16),
                   jax.ShapeDtypeStruct((N, H), jnp.bfloat16)],
    )(op, op, d0, d1, b, nw1, nb1, nw2, nb2, ea, eb1, eb)


VSR = E // 8
EPB = 1000


def _tc_epfin_body(vs_ref, sel_ref, b2_ref, out_ref):
    acc = jnp.dot(vs_ref[...], sel_ref[...], preferred_element_type=jnp.float32)
    out_ref[...] = jax.nn.sigmoid(acc + b2_ref[...])


def _tc_epfin(vs, sel, b2):
    return pl.pallas_call(
        _tc_epfin_body,
        grid=(VSR // EPB,),
        in_specs=[
            pl.BlockSpec((EPB, 128), lambda i: (i, 0)),
            pl.BlockSpec((128, 8), lambda i: (0, 0)),
            pl.BlockSpec((1, 1), lambda i: (0, 0)),
        ],
        out_specs=pl.BlockSpec((EPB, 8), lambda i: (i, 0)),
        out_shape=_f32(VSR, 8),
    )(vs, sel, b2)


def _make_sc_layer():
    def body(h_hbm, s_hbm, d_hbm, esrc_hbm, edst_hbm, denp_hbm, outp_hbm,
             s_v, d_v, src_v, dst_v, den_v, red_v, acc_v, ex_all,
             d80_0, d80_1, rows_0, rows_1, den_sh, out_sh,
             gsem_0, gsem_1, ssem_0, ssem_1):
        c = lax.axis_index("c")
        sid = lax.axis_index("s")
        w = sid * NC + c
        base = w * EPW

        pltpu.sync_copy(s_hbm, s_v)
        pltpu.sync_copy(d_hbm, d_v)
        pltpu.sync_copy(esrc_hbm.at[pl.ds(base, EPW)], src_v)
        pltpu.sync_copy(edst_hbm.at[pl.ds(base, EPW)], dst_v)

        def zero_body(g, _):
            den_v[pl.ds(g * L, L)] = jnp.zeros((L,), jnp.float32)
            return 0
        lax.fori_loop(0, NPAD // L, zero_body, 0)

        def zz_body(r, _):
            for t in range(H // L):
                rows_0[r, pl.ds(t * L, L)] = jnp.zeros((L,), jnp.float32)
            return 0
        lax.fori_loop(0, CHB, zz_body, 0)
        for q in range(SLICE // CHB):
            pltpu.sync_copy(
                rows_0, out_sh.at[pl.ds(sid * SLICE + q * CHB, CHB), :])
        for t in range(SLICE % CHB // 8):
            pass
        if SLICE % CHB:
            pltpu.sync_copy(
                rows_0.at[pl.ds(0, SLICE % CHB), :],
                out_sh.at[pl.ds(sid * SLICE + (SLICE // CHB) * CHB,
                                SLICE % CHB), :])
        plsc.subcore_barrier()

        def e_body(g, _):
            sl = pl.ds(g * L, L)
            s16 = src_v[sl]
            d16 = dst_v[sl]
            e = plsc.load_gather(s_v, [s16]) + plsc.load_gather(d_v, [d16])
            e = jnp.where(e > 0, e, e * 0.2)
            ex = jnp.exp(jnp.minimum(e, 60.0))
            ex_all[sl] = ex
            plsc.addupdate_scatter(den_v, [d16], ex)
            return 0
        lax.fori_loop(0, EPW // L, e_body, 0)

        def start_gather(k, rows_b, gsem_b):
            for j in range(NSUB):
                pltpu.async_copy(
                    h_hbm.at[src_v.at[pl.ds(k * CHB + j * SUB, SUB)]],
                    rows_b.at[pl.ds(j * SUB, SUB), :], gsem_b)

        def process(k, d80_b, rows_b, gsem_b, ssem_b):
            def d_body(g, _):
                j = lax.div(g * L, SUB)
                off = lax.rem(g * L, SUB)
                d80_b[j, pl.ds(off, L)] = dst_v[pl.ds(k * CHB + g * L, L)]
                return 0
            lax.fori_loop(0, CHB // L, d_body, 0)
            for j in range(NSUB):
                pltpu.make_async_copy(
                    h_hbm.at[src_v.at[pl.ds(k * CHB + j * SUB, SUB)]],
                    rows_b.at[pl.ds(j * SUB, SUB), :], gsem_b).wait()

            def sc_body(g, _):
                for j in range(L):
                    av = plsc.load_gather(
                        ex_all,
                        [jnp.full((L,), k * CHB + g * L + j, jnp.int32)])
                    r = g * L + j
                    for t in range(H // L):
                        rows_b[r, pl.ds(t * L, L)] = (
                            rows_b[r, pl.ds(t * L, L)] * av)
                return 0
            lax.fori_loop(0, CHB // L, sc_body, 0)
            for j in range(NSUB):
                pltpu.async_copy(rows_b.at[pl.ds(j * SUB, SUB), :],
                                 out_sh.at[d80_b.at[j]], ssem_b, add=True)

        def wait_scatter(rows_b, d80_b, ssem_b):
            for j in range(NSUB):
                pltpu.make_async_copy(rows_b.at[pl.ds(j * SUB, SUB), :],
                                      out_sh.at[d80_b.at[j]], ssem_b).wait()

        start_gather(0, rows_0, gsem_0)

        def pair_body(p, _):
            k0 = p * 2
            k1 = k0 + 1

            @pl.when(p > 0)
            def _():
                wait_scatter(rows_1, d80_1, ssem_1)
            start_gather(k1, rows_1, gsem_1)
            process(k0, d80_0, rows_0, gsem_0, ssem_0)
            wait_scatter(rows_0, d80_0, ssem_0)

            @pl.when(k1 + 1 < NBCH)
            def _():
                start_gather(k1 + 1, rows_0, gsem_0)
            process(k1, d80_1, rows_1, gsem_1, ssem_1)
            return 0
        lax.fori_loop(0, NBCH // 2, pair_body, 0)
        if NBCH % 2:
            process(NBCH - 1, d80_0, rows_0, gsem_0, ssem_0)
            wait_scatter(rows_0, d80_0, ssem_0)
            wait_scatter(rows_1, d80_1, ssem_1)
        else:
            wait_scatter(rows_1, d80_1, ssem_1)

        plsc.subcore_barrier()

        pltpu.sync_copy(den_v, den_sh.at[sid])
        plsc.subcore_barrier()
        cbase = sid * SLICE

        def zacc_body(g, _):
            acc_v[pl.ds(g * L, L)] = jnp.zeros((L,), jnp.float32)
            return 0
        lax.fori_loop(0, SLICE // L, zacc_body, 0)

        def red_body(t, _):
            pltpu.sync_copy(den_sh.at[t, pl.ds(cbase, SLICE)], red_v)

            def add_body(g, _):
                acc_v[pl.ds(g * L, L)] = (acc_v[pl.ds(g * L, L)]
                                          + red_v[pl.ds(g * L, L)])
                return 0
            lax.fori_loop(0, SLICE // L, add_body, 0)
            return 0
        lax.fori_loop(0, NS, red_body, 0)

        pltpu.sync_copy(acc_v, denp_hbm.at[c, pl.ds(cbase, SLICE)])
        pltpu.sync_copy(out_sh.at[pl.ds(sid * SLICE, SLICE), :],
                        outp_hbm.at[c, pl.ds(sid * SLICE, SLICE), :])

    return pl.kernel(
        body,
        out_type=[_f32(NC, NPAD), _f32(NC, NPAD, H)],
        mesh=_MESH,
        compiler_params=_SC_PARAMS,
        scratch_types=[
            pltpu.VMEM((N,), jnp.float32),
            pltpu.VMEM((N,), jnp.float32),
            pltpu.VMEM((EPW,), jnp.int32),
            pltpu.VMEM((EPW,), jnp.int32),
            pltpu.VMEM((NPAD,), jnp.float32),
            pltpu.VMEM((SLICE,), jnp.float32),
            pltpu.VMEM((SLICE,), jnp.float32),
            pltpu.VMEM((EPW,), jnp.float32),
            pltpu.VMEM((NSUB, SUB), jnp.int32),
            pltpu.VMEM((NSUB, SUB), jnp.int32),
            pltpu.VMEM((CHB, H), jnp.float32),
            pltpu.VMEM((CHB, H), jnp.float32),
            pltpu.VMEM_SHARED((NS, NPAD), jnp.float32),
            pltpu.VMEM_SHARED((NPAD, H), jnp.float32),
            pltpu.SemaphoreType.DMA,
            pltpu.SemaphoreType.DMA,
            pltpu.SemaphoreType.DMA,
            pltpu.SemaphoreType.DMA,
        ],
    )


_sc_layer = _make_sc_layer()


def _make_sc_ep():
    def body(p_hbm, q_hbm, esrc_hbm, edst_hbm, w2_hbm, vs_hbm,
             src_v, dst_v, w2_v, pr_0, qr_0, pr_1, qr_1, vsb_0, vsb_1,
             psem_0, qsem_0, psem_1, qsem_1, wsem_0, wsem_1):
        c = lax.axis_index("c")
        sid = lax.axis_index("s")
        w = sid * NC + c
        base = w * EPW
        vbase = w * (EPW // 8)

        pltpu.sync_copy(w2_hbm, w2_v)
        pltpu.sync_copy(esrc_hbm.at[pl.ds(base, EPW)], src_v)
        pltpu.sync_copy(edst_hbm.at[pl.ds(base, EPW)], dst_v)
        w2u = []
        for t in range(2):
            wa, wb = plsc.unpack(w2_v[pl.ds(t * 2 * L, 2 * L)],
                                 format=plsc.PackFormat.INTERLEAVED)
            w2u += [wa, wb]

        def start_gather(k, pr_b, qr_b, psem_b, qsem_b):
            pltpu.async_copy(p_hbm.at[src_v.at[pl.ds(k * CH, CH)]], pr_b,
                             psem_b)
            pltpu.async_copy(q_hbm.at[dst_v.at[pl.ds(k * CH, CH)]], qr_b,
                             qsem_b)

        def process(k, pr_b, qr_b, psem_b, qsem_b, vsb_b, wsem_b):
            pltpu.make_async_copy(
                p_hbm.at[src_v.at[pl.ds(k * CH, CH)]], pr_b, psem_b).wait()
            pltpu.make_async_copy(
                q_hbm.at[dst_v.at[pl.ds(k * CH, CH)]], qr_b, qsem_b).wait()
            def dot_body(i, _):
                acc = jnp.zeros((L,), jnp.float32)
                for t in range(2):
                    sl = pl.ds(t * 2 * L, 2 * L)
                    pa, pb = plsc.unpack(pr_b[i, sl],
                                         format=plsc.PackFormat.INTERLEAVED)
                    qa, qb = plsc.unpack(qr_b[i, sl],
                                         format=plsc.PackFormat.INTERLEAVED)
                    acc = acc + jnp.maximum(pa + qa, 0.0) * w2u[2 * t]
                    acc = acc + jnp.maximum(pb + qb, 0.0) * w2u[2 * t + 1]
                vsb_b[lax.div(i, 8), pl.ds(lax.rem(i, 8) * L, L)] = acc
                return 0
            lax.fori_loop(0, CH, dot_body, 0)
            pltpu.async_copy(
                vsb_b, vs_hbm.at[pl.ds(vbase + k * (CH // 8), CH // 8), :],
                wsem_b)

        def wait_write(k, vsb_b, wsem_b):
            pltpu.make_async_copy(
                vsb_b, vs_hbm.at[pl.ds(vbase + k * (CH // 8), CH // 8), :],
                wsem_b).wait()

        start_gather(0, pr_0, qr_0, psem_0, qsem_0)

        def pair_body(p, _):
            k0 = p * 2
            k1 = k0 + 1
            start_gather(k1, pr_1, qr_1, psem_1, qsem_1)

            @pl.when(p > 0)
            def _():
                wait_write(k0 - 2, vsb_0, wsem_0)
            process(k0, pr_0, qr_0, psem_0, qsem_0, vsb_0, wsem_0)

            @pl.when(k1 + 1 < NCHUNK)
            def _():
                start_gather(k1 + 1, pr_0, qr_0, psem_0, qsem_0)

            @pl.when(p > 0)
            def _():
                wait_write(k1 - 2, vsb_1, wsem_1)
            process(k1, pr_1, qr_1, psem_1, qsem_1, vsb_1, wsem_1)
            return 0
        lax.fori_loop(0, NCHUNK // 2, pair_body, 0)
        if NCHUNK % 2:
            k = NCHUNK - 1
            wait_write(k - 2, vsb_0, wsem_0)
            process(k, pr_0, qr_0, psem_0, qsem_0, vsb_0, wsem_0)
            wait_write(k, vsb_0, wsem_0)
            wait_write(k - 1, vsb_1, wsem_1)
        else:
            wait_write(NCHUNK - 2, vsb_0, wsem_0)
            wait_write(NCHUNK - 1, vsb_1, wsem_1)

    return pl.kernel(
        body,
        out_type=[_f32(VSR, 128)],
        mesh=_MESH,
        compiler_params=_SC_PARAMS,
        scratch_types=[
            pltpu.VMEM((EPW,), jnp.int32),
            pltpu.VMEM((EPW,), jnp.int32),
            pltpu.VMEM((H,), jnp.bfloat16),
            pltpu.VMEM((CH, H), jnp.bfloat16),
            pltpu.VMEM((CH, H), jnp.bfloat16),
            pltpu.VMEM((CH, H), jnp.bfloat16),
            pltpu.VMEM((CH, H), jnp.bfloat16),
            pltpu.VMEM((CH // 8, 128), jnp.float32),
            pltpu.VMEM((CH // 8, 128), jnp.float32),
            pltpu.SemaphoreType.DMA,
            pltpu.SemaphoreType.DMA,
            pltpu.SemaphoreType.DMA,
            pltpu.SemaphoreType.DMA,
            pltpu.SemaphoreType.DMA,
            pltpu.SemaphoreType.DMA,
        ],
    )


_sc_ep = _make_sc_ep()


def kernel(x, edge_index, W1, as1, ad1, b1, W2, as2, ad2, b2,
           W3, as3, ad3, b3, npW1, npb1, npW2, npb2, epW1, epb1, epW2, epb2):
    zpad = jnp.zeros((H, 6), jnp.float32)
    A1 = jnp.concatenate([as1[:, None], ad1[:, None], zpad], axis=1)
    A2 = jnp.concatenate([as2[:, None], ad2[:, None], zpad], axis=1)
    A3 = jnp.concatenate([as3[:, None], ad3[:, None], zpad], axis=1)

    esrc = edge_index[0]
    edst = edge_index[1]

    h1, sd1 = _tc_encode(x, W1, A1)
    denp1, outp1 = _sc_layer(h1, sd1[:, 0], sd1[:, 1], esrc, edst)

    h2, sd2 = _tc_mid(outp1, denp1[0][:, None], denp1[1][:, None],
                      b1[None, :], W2, A2)
    denp2, outp2 = _sc_layer(h2, sd2[:, 0], sd2[:, 1], esrc, edst)

    h3, sd3 = _tc_mid(outp2, denp2[0][:, None], denp2[1][:, None],
                      b2[None, :], W3, A3)
    denp3, outp3 = _sc_layer(h3, sd3[:, 0], sd3[:, 1], esrc, edst)

    nf, P, Q = _tc_head(outp3, denp3[0][:, None],
                        denp3[1][:, None], b3[None, :],
                        npW1, npb1[None, :], npW2, npb2[None, :],
                        epW1[:H], epb1[None, :], epW1[H:])

    (vs,) = _sc_ep(P, Q, esrc, edst, epW2[:, 0].astype(jnp.bfloat16))

    lane = jnp.arange(128) // 16
    grp = jnp.arange(8)
    sel = (lane[:, None] == grp[None, :]).astype(jnp.float32)
    ep8 = _tc_epfin(vs, sel, epb2[None, :])
    return nf, ep8.reshape(E, 1)
